# baseline (speedup 1.0000x reference)
import functools

import jax
import jax.numpy as jnp
from jax import lax
from jax.experimental import pallas as pl
from jax.experimental.pallas import tpu as pltpu
from jax.experimental.pallas import tpu_sc as plsc

K_TOP = 64
KEYS = 100000
NQ = 1024
KBLK = 2048
SUB = 128
NB = 784
KPAD = NB * SUB
NSTEP = KPAD // KBLK
CPB = KBLK // SUB

NEG = -3.0
BIG = 2 ** 30


def _row_norm(x):
    parts = [x[:, j:j + 1] * x[:, j:j + 1] for j in range(16)]
    d = 8
    while d >= 1:
        parts = [parts[i] + parts[i + d] for i in range(d)]
        d //= 2
    return jnp.maximum(jnp.sqrt(parts[0]), 1e-12)


def _p1_kernel(z_ref, te_ref, s3_ref, bm_ref):
    i = pl.program_id(0)
    z = z_ref[...]
    qn = z / _row_norm(z)
    t = te_ref[...]
    tn = t / _row_norm(t)
    for c in range(CPB):
        tn_c = tn[c * SUB:(c + 1) * SUB, :]
        s = jax.lax.dot_general(qn, tn_c, (((1,), (1,)), ((), ())),
                                preferred_element_type=jnp.float32)
        key_id = (i * KBLK + c * SUB
                  + jax.lax.broadcasted_iota(jnp.int32, s.shape, 1))
        s = jnp.where(key_id < KEYS, s, -2.0)
        s3_ref[c] = s
        bm_ref[0, :, c:c + 1] = jnp.max(s, axis=1, keepdims=True)


def _p2_kernel(bm_ref, bidx_ref, thr_ref, scr_ref):
    scr_ref[...] = bm_ref[...]
    rows = bm_ref.shape[0]
    col = jax.lax.broadcasted_iota(jnp.int32, (rows, NB), 1)
    col64 = jax.lax.broadcasted_iota(jnp.int32, (rows, K_TOP), 1)

    def body(r, carry):
        acc, thr = carry
        data = scr_ref[...]
        m = jnp.max(data, axis=1, keepdims=True)
        sel = jnp.min(jnp.where(data == m, col, BIG), axis=1, keepdims=True)
        scr_ref[...] = jnp.where(col == sel, NEG, data)
        return (jnp.where(col64 == r, sel, acc),
                jnp.where(r == K_TOP - 1, m, thr))

    picked, thr = jax.lax.fori_loop(
        0, K_TOP, body,
        (jnp.zeros((rows, K_TOP), jnp.int32), jnp.zeros((rows, 1), jnp.float32)))
    thr_ref[...] = thr

    def body2(r, carry):
        data, acc = carry
        m = jnp.min(data, axis=1, keepdims=True)
        data = jnp.where(data == m, BIG, data)
        return data, jnp.where(col64 == r, m, acc)

    _, srt = jax.lax.fori_loop(0, K_TOP, body2,
                               (picked, jnp.zeros((rows, K_TOP), jnp.int32)))
    bidx_ref[...] = srt


CAP = 512


def _sc_gather_filter(s3flat, bidx_flat, thr16, gb16):
    info = plsc.get_sparse_core_info()
    nw = info.num_cores * info.num_subcores
    per_w = (NQ * K_TOP) // nw
    q_per_w = NQ // nw
    chunk = 256
    mesh = plsc.VectorSubcoreMesh(core_axis_name="c", subcore_axis_name="s")

    @functools.partial(
        pl.kernel, mesh=mesh,
        compiler_params=pltpu.CompilerParams(needs_layout_passes=False),
        out_type=[jax.ShapeDtypeStruct((NQ, CAP), jnp.float32),
                  jax.ShapeDtypeStruct((NQ, CAP), jnp.int32)],
        scratch_types=[
            pltpu.VMEM((per_w,), jnp.int32),
            pltpu.VMEM((chunk, SUB), jnp.float32),
            pltpu.VMEM((CAP,), jnp.float32),
            pltpu.VMEM((CAP,), jnp.int32),
            pltpu.VMEM((q_per_w * 16,), jnp.float32),
            pltpu.VMEM((per_w * 16,), jnp.int32),
            pltpu.SemaphoreType.DMA,
        ],
    )
    def k(s3_hbm, bidx_hbm, thr_hbm, gb_hbm, outv_hbm, outg_hbm,
          idx_v, rows_v, valb, gb, thr_v, gb16_v, sem):
        wid = lax.axis_index("s") * info.num_cores + lax.axis_index("c")
        base = wid * per_w
        pltpu.sync_copy(bidx_hbm.at[pl.ds(base, per_w)], idx_v)
        pltpu.sync_copy(thr_hbm.at[pl.ds(wid * q_per_w * 16, q_per_w * 16)],
                        thr_v)
        pltpu.sync_copy(gb_hbm.at[pl.ds(base * 16, per_w * 16)], gb16_v)

        def to_rows(kk, _):
            off = pl.multiple_of(kk * 16, 16)
            q = wid * q_per_w + (kk >> 2)
            idx_v[pl.ds(off, 16)] = idx_v[pl.ds(off, 16)] * NQ + q
            return 0

        lax.fori_loop(0, per_w // 16, to_rows, 0)

        for qc in range(per_w // chunk):
            pltpu.async_copy(
                s3_hbm.at[idx_v.at[pl.ds(qc * chunk, chunk)]], rows_v, sem
            ).wait()
            for ql in range(4):
                q_local = qc * 4 + ql
                tq = thr_v[pl.ds(q_local * 16, 16)]
                for j in range(CAP // 16):
                    valb[pl.ds(j * 16, 16)] = jnp.full((16,), NEG, jnp.float32)

                def row_body(r, ptr):
                    lane = jax.lax.iota(jnp.int32, 16)
                    rowpos = ql * 64 + r
                    goff = pl.multiple_of((qc * chunk + rowpos) * 16, 16)
                    gbase = gb16_v[pl.ds(goff, 16)]

                    def hit(p, v, g, mask):
                        pos = p + plsc.cumsum(
                            jnp.where(mask, 1, 0).astype(jnp.int32)) - 1
                        pos = jnp.minimum(pos, CAP - 1)
                        plsc.store_scatter(valb, [pos], v, mask=mask)
                        plsc.store_scatter(gb, [pos], g, mask=mask)
                        return p + plsc.all_reduce_population_count(mask)

                    def vreg_body(l, p):
                        v = rows_v[rowpos, pl.ds(l * 16, 16)]
                        mask = v >= tq
                        return lax.cond(
                            jnp.any(mask),
                            lambda: hit(p, v, gbase + l * 16 + lane, mask),
                            lambda: p)

                    return lax.fori_loop(0, SUB // 16, vreg_body, ptr)

                lax.fori_loop(0, K_TOP, row_body,
                              jnp.zeros((16,), jnp.int32))
                qg = wid * q_per_w + q_local
                pltpu.sync_copy(valb, outv_hbm.at[qg])
                pltpu.sync_copy(gb, outg_hbm.at[qg])

    return k(s3flat, bidx_flat, thr16.reshape(NQ * 16), gb16.reshape(NQ * K_TOP * 16))


def _sc_gather(s3flat, bidx_flat):
    info = plsc.get_sparse_core_info()
    nw = info.num_cores * info.num_subcores
    per_w = (NQ * K_TOP) // nw
    q_per_w = NQ // nw
    chunk = 256
    nchunk = per_w // chunk
    mesh = plsc.VectorSubcoreMesh(core_axis_name="c", subcore_axis_name="s")

    @functools.partial(
        pl.kernel, mesh=mesh,
        out_type=jax.ShapeDtypeStruct((NQ * K_TOP, SUB), jnp.float32),
        scratch_types=[
            pltpu.VMEM((per_w,), jnp.int32),
            pltpu.VMEM((chunk, SUB), jnp.float32),
            pltpu.SemaphoreType.DMA,
        ],
    )
    def k(s3_hbm, bidx_hbm, out_hbm, idx_v, rows_v, sem):
        wid = lax.axis_index("s") * info.num_cores + lax.axis_index("c")
        base = wid * per_w
        pltpu.sync_copy(bidx_hbm.at[pl.ds(base, per_w)], idx_v)

        def to_rows(kk, _):
            off = pl.multiple_of(kk * 16, 16)
            q = wid * q_per_w + (kk >> 2)
            idx_v[pl.ds(off, 16)] = idx_v[pl.ds(off, 16)] * NQ + q
            return 0

        lax.fori_loop(0, per_w // 16, to_rows, 0)
        for c in range(nchunk):
            pltpu.async_copy(
                s3_hbm.at[idx_v.at[pl.ds(c * chunk, chunk)]], rows_v, sem
            ).wait()
            pltpu.sync_copy(rows_v, out_hbm.at[pl.ds(base + c * chunk, chunk)])

    return k(s3flat, bidx_flat)


def _p4_kernel(cand_ref, g_ref, val_ref, idx_ref, scr_ref):
    scr_ref[...] = cand_ref[...]
    g = g_ref[...]
    rows = cand_ref.shape[0]
    col64 = jax.lax.broadcasted_iota(jnp.int32, (rows, K_TOP), 1)

    def body(r, carry):
        vacc, iacc = carry
        data = scr_ref[...]
        m = jnp.max(data, axis=1, keepdims=True)
        gsel = jnp.min(jnp.where(data == m, g, BIG), axis=1, keepdims=True)
        scr_ref[...] = jnp.where(g == gsel, NEG, data)
        vacc = jnp.where(col64 == r, m, vacc)
        iacc = jnp.where(col64 == r, gsel, iacc)
        return vacc, iacc

    vacc, iacc = jax.lax.fori_loop(
        0, K_TOP, body,
        (jnp.zeros((rows, K_TOP), jnp.float32), jnp.zeros((rows, K_TOP), jnp.int32)))
    val_ref[...] = vacc
    idx_ref[...] = iacc


def kernel(z_cell, type_embeddings):
    te = jnp.pad(type_embeddings, ((0, KPAD - KEYS), (0, 0)))

    s3, bm = pl.pallas_call(
        _p1_kernel,
        grid=(NSTEP,),
        in_specs=[pl.BlockSpec((NQ, 16), lambda i: (0, 0)),
                  pl.BlockSpec((KBLK, 16), lambda i: (i, 0))],
        out_specs=[pl.BlockSpec((CPB, NQ, SUB), lambda i: (i, 0, 0)),
                   pl.BlockSpec((1, NQ, CPB), lambda i: (i, 0, 0))],
        out_shape=[jax.ShapeDtypeStruct((NB, NQ, SUB), jnp.float32),
                   jax.ShapeDtypeStruct((NSTEP, NQ, CPB), jnp.float32)],
    )(z_cell, te)
    bm = bm.transpose(1, 0, 2).reshape(NQ, NB)

    qrows = 256
    bidx, thr = pl.pallas_call(
        _p2_kernel,
        grid=(NQ // qrows,),
        in_specs=[pl.BlockSpec((qrows, NB), lambda i: (i, 0))],
        out_specs=[pl.BlockSpec((qrows, K_TOP), lambda i: (i, 0)),
                   pl.BlockSpec((qrows, 1), lambda i: (i, 0))],
        out_shape=[jax.ShapeDtypeStruct((NQ, K_TOP), jnp.int32),
                   jax.ShapeDtypeStruct((NQ, 1), jnp.float32)],
        scratch_shapes=[pltpu.VMEM((qrows, NB), jnp.float32)],
    )(bm)

    thr16 = jnp.broadcast_to(thr, (NQ, 16))
    gb16 = jnp.broadcast_to((bidx * SUB).reshape(NQ * K_TOP, 1),
                            (NQ * K_TOP, 16))
    cv, cg = _sc_gather_filter(s3.reshape(NB * NQ, SUB),
                               bidx.reshape(NQ * K_TOP), thr16, gb16)

    prows = 256
    vals, idxs = pl.pallas_call(
        _p4_kernel,
        grid=(NQ // prows,),
        in_specs=[pl.BlockSpec((prows, CAP), lambda i: (i, 0)),
                  pl.BlockSpec((prows, CAP), lambda i: (i, 0))],
        out_specs=[pl.BlockSpec((prows, K_TOP), lambda i: (i, 0)),
                   pl.BlockSpec((prows, K_TOP), lambda i: (i, 0))],
        out_shape=[jax.ShapeDtypeStruct((NQ, K_TOP), jnp.float32),
                   jax.ShapeDtypeStruct((NQ, K_TOP), jnp.int32)],
        scratch_shapes=[pltpu.VMEM((prows, CAP), jnp.float32)],
    )(cv, cg)

    return vals, idxs


# speedup vs baseline: 1.3168x; 1.3168x over previous
import functools

import jax
import jax.numpy as jnp
from jax import lax
from jax.experimental import pallas as pl
from jax.experimental.pallas import tpu as pltpu
from jax.experimental.pallas import tpu_sc as plsc

K_TOP = 64
KEYS = 100000
NQ = 1024
KBLK = 2048
SUB = 128
NB = 784
KPAD = NB * SUB
NSTEP = KPAD // KBLK
CPB = KBLK // SUB

NEG = -3.0
BIG = 2 ** 30


def _row_norm(x):
    parts = [x[:, j:j + 1] * x[:, j:j + 1] for j in range(16)]
    d = 8
    while d >= 1:
        parts = [parts[i] + parts[i + d] for i in range(d)]
        d //= 2
    return jnp.maximum(jnp.sqrt(parts[0]), 1e-12)


def _p1_kernel(z_ref, te_ref, s3_ref, bm_ref):
    i = pl.program_id(0)
    z = z_ref[...]
    qn = z / _row_norm(z)
    t = te_ref[...]
    tn = t / _row_norm(t)
    for c in range(CPB):
        tn_c = tn[c * SUB:(c + 1) * SUB, :]
        s = jax.lax.dot_general(qn, tn_c, (((1,), (1,)), ((), ())),
                                preferred_element_type=jnp.float32)
        key_id = (i * KBLK + c * SUB
                  + jax.lax.broadcasted_iota(jnp.int32, s.shape, 1))
        s = jnp.where(key_id < KEYS, s, -2.0)
        s3_ref[c] = s
        bm_ref[0, :, c:c + 1] = jnp.max(s, axis=1, keepdims=True)


def _p2_kernel(bm_ref, bidx_ref, thr_ref, scr_ref):
    scr_ref[...] = bm_ref[...]
    rows = bm_ref.shape[0]
    col = jax.lax.broadcasted_iota(jnp.int32, (rows, NB), 1)
    col64 = jax.lax.broadcasted_iota(jnp.int32, (rows, K_TOP), 1)

    def body(r, carry):
        acc, thr = carry
        data = scr_ref[...]
        m = jnp.max(data, axis=1, keepdims=True)
        sel = jnp.min(jnp.where(data == m, col, BIG), axis=1, keepdims=True)
        scr_ref[...] = jnp.where(col == sel, NEG, data)
        return (jnp.where(col64 == r, sel, acc),
                jnp.where(r == K_TOP - 1, m, thr))

    picked, thr = jax.lax.fori_loop(
        0, K_TOP, body,
        (jnp.zeros((rows, K_TOP), jnp.int32), jnp.zeros((rows, 1), jnp.float32)))
    thr_ref[...] = thr

    def body2(r, carry):
        data, acc = carry
        m = jnp.min(data, axis=1, keepdims=True)
        data = jnp.where(data == m, BIG, data)
        return data, jnp.where(col64 == r, m, acc)

    _, srt = jax.lax.fori_loop(0, K_TOP, body2,
                               (picked, jnp.zeros((rows, K_TOP), jnp.int32)))
    bidx_ref[...] = srt


CAP = 512


def _sc_gather_filter(s3flat, bidx_flat, thr16, gb16):
    info = plsc.get_sparse_core_info()
    nw = info.num_cores * info.num_subcores
    per_w = (NQ * K_TOP) // nw
    q_per_w = NQ // nw
    chunk = 256
    mesh = plsc.VectorSubcoreMesh(core_axis_name="c", subcore_axis_name="s")

    @functools.partial(
        pl.kernel, mesh=mesh,
        compiler_params=pltpu.CompilerParams(needs_layout_passes=False),
        out_type=[jax.ShapeDtypeStruct((NQ, CAP), jnp.float32),
                  jax.ShapeDtypeStruct((NQ, CAP), jnp.int32)],
        scratch_types=[
            pltpu.VMEM((per_w,), jnp.int32),
            pltpu.VMEM((chunk, SUB), jnp.float32),
            pltpu.VMEM((CAP,), jnp.float32),
            pltpu.VMEM((CAP,), jnp.int32),
            pltpu.VMEM((q_per_w * 16,), jnp.float32),
            pltpu.VMEM((per_w * 16,), jnp.int32),
            pltpu.SemaphoreType.DMA,
        ],
    )
    def k(s3_hbm, bidx_hbm, thr_hbm, gb_hbm, outv_hbm, outg_hbm,
          idx_v, rows_v, valb, gb, thr_v, gb16_v, sem):
        wid = lax.axis_index("s") * info.num_cores + lax.axis_index("c")
        base = wid * per_w
        pltpu.sync_copy(bidx_hbm.at[pl.ds(base, per_w)], idx_v)
        pltpu.sync_copy(thr_hbm.at[pl.ds(wid * q_per_w * 16, q_per_w * 16)],
                        thr_v)
        pltpu.sync_copy(gb_hbm.at[pl.ds(base * 16, per_w * 16)], gb16_v)

        def to_rows(kk, _):
            off = pl.multiple_of(kk * 16, 16)
            q = wid * q_per_w + (kk >> 2)
            idx_v[pl.ds(off, 16)] = idx_v[pl.ds(off, 16)] * NQ + q
            return 0

        lax.fori_loop(0, per_w // 16, to_rows, 0)

        for qc in range(per_w // chunk):
            pltpu.async_copy(
                s3_hbm.at[idx_v.at[pl.ds(qc * chunk, chunk)]], rows_v, sem
            ).wait()
            for ql in range(4):
                q_local = qc * 4 + ql
                tq = thr_v[pl.ds(q_local * 16, 16)]
                for j in range(CAP // 16):
                    valb[pl.ds(j * 16, 16)] = jnp.full((16,), NEG, jnp.float32)

                def row_body(r, ptr):
                    lane = jax.lax.iota(jnp.int32, 16)
                    rowpos = ql * 64 + r
                    goff = pl.multiple_of((qc * chunk + rowpos) * 16, 16)
                    gbase = gb16_v[pl.ds(goff, 16)]

                    def hit(p, v, g, mask):
                        pos = p + plsc.cumsum(
                            jnp.where(mask, 1, 0).astype(jnp.int32)) - 1
                        pos = jnp.minimum(pos, CAP - 1)
                        plsc.store_scatter(valb, [pos], v, mask=mask)
                        plsc.store_scatter(gb, [pos], g, mask=mask)
                        return p + plsc.all_reduce_population_count(mask)

                    p = ptr
                    for l in range(SUB // 16):
                        v = rows_v[rowpos, pl.ds(l * 16, 16)]
                        mask = v >= tq
                        p = hit(p, v, gbase + l * 16 + lane, mask)
                    return p

                lax.fori_loop(0, K_TOP, row_body,
                              jnp.zeros((16,), jnp.int32))
                qg = wid * q_per_w + q_local
                pltpu.sync_copy(valb, outv_hbm.at[qg])
                pltpu.sync_copy(gb, outg_hbm.at[qg])

    return k(s3flat, bidx_flat, thr16.reshape(NQ * 16), gb16.reshape(NQ * K_TOP * 16))


def _sc_gather(s3flat, bidx_flat):
    info = plsc.get_sparse_core_info()
    nw = info.num_cores * info.num_subcores
    per_w = (NQ * K_TOP) // nw
    q_per_w = NQ // nw
    chunk = 256
    nchunk = per_w // chunk
    mesh = plsc.VectorSubcoreMesh(core_axis_name="c", subcore_axis_name="s")

    @functools.partial(
        pl.kernel, mesh=mesh,
        out_type=jax.ShapeDtypeStruct((NQ * K_TOP, SUB), jnp.float32),
        scratch_types=[
            pltpu.VMEM((per_w,), jnp.int32),
            pltpu.VMEM((chunk, SUB), jnp.float32),
            pltpu.SemaphoreType.DMA,
        ],
    )
    def k(s3_hbm, bidx_hbm, out_hbm, idx_v, rows_v, sem):
        wid = lax.axis_index("s") * info.num_cores + lax.axis_index("c")
        base = wid * per_w
        pltpu.sync_copy(bidx_hbm.at[pl.ds(base, per_w)], idx_v)

        def to_rows(kk, _):
            off = pl.multiple_of(kk * 16, 16)
            q = wid * q_per_w + (kk >> 2)
            idx_v[pl.ds(off, 16)] = idx_v[pl.ds(off, 16)] * NQ + q
            return 0

        lax.fori_loop(0, per_w // 16, to_rows, 0)
        for c in range(nchunk):
            pltpu.async_copy(
                s3_hbm.at[idx_v.at[pl.ds(c * chunk, chunk)]], rows_v, sem
            ).wait()
            pltpu.sync_copy(rows_v, out_hbm.at[pl.ds(base + c * chunk, chunk)])

    return k(s3flat, bidx_flat)


def _p4_kernel(cand_ref, g_ref, val_ref, idx_ref, scr_ref):
    scr_ref[...] = cand_ref[...]
    g = g_ref[...]
    rows = cand_ref.shape[0]
    col64 = jax.lax.broadcasted_iota(jnp.int32, (rows, K_TOP), 1)

    def body(r, carry):
        vacc, iacc = carry
        data = scr_ref[...]
        m = jnp.max(data, axis=1, keepdims=True)
        gsel = jnp.min(jnp.where(data == m, g, BIG), axis=1, keepdims=True)
        scr_ref[...] = jnp.where(g == gsel, NEG, data)
        vacc = jnp.where(col64 == r, m, vacc)
        iacc = jnp.where(col64 == r, gsel, iacc)
        return vacc, iacc

    vacc, iacc = jax.lax.fori_loop(
        0, K_TOP, body,
        (jnp.zeros((rows, K_TOP), jnp.float32), jnp.zeros((rows, K_TOP), jnp.int32)))
    val_ref[...] = vacc
    idx_ref[...] = iacc


def kernel(z_cell, type_embeddings):
    te = jnp.pad(type_embeddings, ((0, KPAD - KEYS), (0, 0)))

    s3, bm = pl.pallas_call(
        _p1_kernel,
        grid=(NSTEP,),
        in_specs=[pl.BlockSpec((NQ, 16), lambda i: (0, 0)),
                  pl.BlockSpec((KBLK, 16), lambda i: (i, 0))],
        out_specs=[pl.BlockSpec((CPB, NQ, SUB), lambda i: (i, 0, 0)),
                   pl.BlockSpec((1, NQ, CPB), lambda i: (i, 0, 0))],
        out_shape=[jax.ShapeDtypeStruct((NB, NQ, SUB), jnp.float32),
                   jax.ShapeDtypeStruct((NSTEP, NQ, CPB), jnp.float32)],
    )(z_cell, te)
    bm = bm.transpose(1, 0, 2).reshape(NQ, NB)

    qrows = 256
    bidx, thr = pl.pallas_call(
        _p2_kernel,
        grid=(NQ // qrows,),
        in_specs=[pl.BlockSpec((qrows, NB), lambda i: (i, 0))],
        out_specs=[pl.BlockSpec((qrows, K_TOP), lambda i: (i, 0)),
                   pl.BlockSpec((qrows, 1), lambda i: (i, 0))],
        out_shape=[jax.ShapeDtypeStruct((NQ, K_TOP), jnp.int32),
                   jax.ShapeDtypeStruct((NQ, 1), jnp.float32)],
        scratch_shapes=[pltpu.VMEM((qrows, NB), jnp.float32)],
    )(bm)

    thr16 = jnp.broadcast_to(thr, (NQ, 16))
    gb16 = jnp.broadcast_to((bidx * SUB).reshape(NQ * K_TOP, 1),
                            (NQ * K_TOP, 16))
    cv, cg = _sc_gather_filter(s3.reshape(NB * NQ, SUB),
                               bidx.reshape(NQ * K_TOP), thr16, gb16)

    prows = 256
    vals, idxs = pl.pallas_call(
        _p4_kernel,
        grid=(NQ // prows,),
        in_specs=[pl.BlockSpec((prows, CAP), lambda i: (i, 0)),
                  pl.BlockSpec((prows, CAP), lambda i: (i, 0))],
        out_specs=[pl.BlockSpec((prows, K_TOP), lambda i: (i, 0)),
                   pl.BlockSpec((prows, K_TOP), lambda i: (i, 0))],
        out_shape=[jax.ShapeDtypeStruct((NQ, K_TOP), jnp.float32),
                   jax.ShapeDtypeStruct((NQ, K_TOP), jnp.int32)],
        scratch_shapes=[pltpu.VMEM((prows, CAP), jnp.float32)],
    )(cv, cg)

    return vals, idxs


# speedup vs baseline: 1.3327x; 1.0120x over previous
import functools

import jax
import jax.numpy as jnp
from jax import lax
from jax.experimental import pallas as pl
from jax.experimental.pallas import tpu as pltpu
from jax.experimental.pallas import tpu_sc as plsc

K_TOP = 64
KEYS = 100000
NQ = 1024
KBLK = 2048
SUB = 128
NB = 784
KPAD = NB * SUB
NSTEP = KPAD // KBLK
CPB = KBLK // SUB

NEG = -3.0
BIG = 2 ** 30


def _row_norm(x):
    parts = [x[:, j:j + 1] * x[:, j:j + 1] for j in range(16)]
    d = 8
    while d >= 1:
        parts = [parts[i] + parts[i + d] for i in range(d)]
        d //= 2
    return jnp.maximum(jnp.sqrt(parts[0]), 1e-12)


def _p1_kernel(z_ref, te_ref, s3_ref, bm_ref):
    i = pl.program_id(0)
    z = z_ref[...]
    qn = z / _row_norm(z)
    t = te_ref[...]
    tn = t / _row_norm(t)
    for c in range(CPB):
        tn_c = tn[c * SUB:(c + 1) * SUB, :]
        s = jax.lax.dot_general(qn, tn_c, (((1,), (1,)), ((), ())),
                                preferred_element_type=jnp.float32)
        key_id = (i * KBLK + c * SUB
                  + jax.lax.broadcasted_iota(jnp.int32, s.shape, 1))
        s = jnp.where(key_id < KEYS, s, -2.0)
        s3_ref[c] = s
        bm_ref[0, :, c:c + 1] = jnp.max(s, axis=1, keepdims=True)


def _p2_kernel(bm_ref, bidx_ref, thr_ref, scr_ref):
    scr_ref[...] = bm_ref[...]
    rows = bm_ref.shape[0]
    col = jax.lax.broadcasted_iota(jnp.int32, (rows, NB), 1)
    col64 = jax.lax.broadcasted_iota(jnp.int32, (rows, K_TOP), 1)

    def body(r, carry):
        acc, thr = carry
        data = scr_ref[...]
        m = jnp.max(data, axis=1, keepdims=True)
        sel = jnp.min(jnp.where(data == m, col, BIG), axis=1, keepdims=True)
        scr_ref[...] = jnp.where(col == sel, NEG, data)
        return (jnp.where(col64 == r, sel, acc),
                jnp.where(r == K_TOP - 1, m, thr))

    picked, thr = jax.lax.fori_loop(
        0, K_TOP, body,
        (jnp.zeros((rows, K_TOP), jnp.int32), jnp.zeros((rows, 1), jnp.float32)))
    thr_ref[...] = thr

    def body2(r, carry):
        data, acc = carry
        m = jnp.min(data, axis=1, keepdims=True)
        data = jnp.where(data == m, BIG, data)
        return data, jnp.where(col64 == r, m, acc)

    _, srt = jax.lax.fori_loop(0, K_TOP, body2,
                               (picked, jnp.zeros((rows, K_TOP), jnp.int32)))
    bidx_ref[...] = srt


CAP = 512


def _sc_gather_filter(s3flat, bidx_flat, thr16, gb16):
    info = plsc.get_sparse_core_info()
    nw = info.num_cores * info.num_subcores
    per_w = (NQ * K_TOP) // nw
    q_per_w = NQ // nw
    chunk = 256
    mesh = plsc.VectorSubcoreMesh(core_axis_name="c", subcore_axis_name="s")

    @functools.partial(
        pl.kernel, mesh=mesh,
        compiler_params=pltpu.CompilerParams(needs_layout_passes=False),
        out_type=[jax.ShapeDtypeStruct((NQ, CAP), jnp.float32),
                  jax.ShapeDtypeStruct((NQ, CAP), jnp.int32)],
        scratch_types=[
            pltpu.VMEM((per_w,), jnp.int32),
            pltpu.VMEM((chunk, SUB), jnp.float32),
            pltpu.VMEM((CAP,), jnp.float32),
            pltpu.VMEM((CAP,), jnp.int32),
            pltpu.VMEM((q_per_w * 16,), jnp.float32),
            pltpu.VMEM((per_w * 16,), jnp.int32),
            pltpu.SemaphoreType.DMA,
        ],
    )
    def k(s3_hbm, bidx_hbm, thr_hbm, gb_hbm, outv_hbm, outg_hbm,
          idx_v, rows_v, valb, gb, thr_v, gb16_v, sem):
        wid = lax.axis_index("s") * info.num_cores + lax.axis_index("c")
        base = wid * per_w
        pltpu.sync_copy(bidx_hbm.at[pl.ds(base, per_w)], idx_v)
        pltpu.sync_copy(thr_hbm.at[pl.ds(wid * q_per_w * 16, q_per_w * 16)],
                        thr_v)
        pltpu.sync_copy(gb_hbm.at[pl.ds(base * 16, per_w * 16)], gb16_v)

        def to_rows(kk, _):
            off = pl.multiple_of(kk * 16, 16)
            q = wid * q_per_w + (kk >> 2)
            idx_v[pl.ds(off, 16)] = idx_v[pl.ds(off, 16)] * NQ + q
            return 0

        lax.fori_loop(0, per_w // 16, to_rows, 0)

        for qc in range(per_w // chunk):
            pltpu.async_copy(
                s3_hbm.at[idx_v.at[pl.ds(qc * chunk, chunk)]], rows_v, sem
            ).wait()
            for ql in range(4):
                q_local = qc * 4 + ql
                tq = thr_v[pl.ds(q_local * 16, 16)]
                for j in range(CAP // 16):
                    valb[pl.ds(j * 16, 16)] = jnp.full((16,), NEG, jnp.float32)

                lane = jax.lax.iota(jnp.int32, 16)
                ones = jnp.ones((16,), jnp.int32)
                goffs = [l * 16 + lane for l in range(SUB // 16)]

                def row_body(r, ptr):
                    rowpos = ql * 64 + r
                    goff = pl.multiple_of((qc * chunk + rowpos) * 16, 16)
                    gbase = gb16_v[pl.ds(goff, 16)]

                    def hit(p, v, g, mask):
                        pos = p + plsc.cumsum(ones, mask=mask) - 1
                        pos = jnp.minimum(pos, CAP - 1)
                        plsc.store_scatter(valb, [pos], v, mask=mask)
                        plsc.store_scatter(gb, [pos], g, mask=mask)
                        return p + plsc.all_reduce_population_count(mask)

                    p = ptr
                    for l in range(SUB // 16):
                        v = rows_v[rowpos, pl.ds(l * 16, 16)]
                        mask = v >= tq
                        p = hit(p, v, gbase + goffs[l], mask)
                    return p

                lax.fori_loop(0, K_TOP, row_body,
                              jnp.zeros((16,), jnp.int32))
                qg = wid * q_per_w + q_local
                pltpu.sync_copy(valb, outv_hbm.at[qg])
                pltpu.sync_copy(gb, outg_hbm.at[qg])

    return k(s3flat, bidx_flat, thr16.reshape(NQ * 16), gb16.reshape(NQ * K_TOP * 16))


def _sc_gather(s3flat, bidx_flat):
    info = plsc.get_sparse_core_info()
    nw = info.num_cores * info.num_subcores
    per_w = (NQ * K_TOP) // nw
    q_per_w = NQ // nw
    chunk = 256
    nchunk = per_w // chunk
    mesh = plsc.VectorSubcoreMesh(core_axis_name="c", subcore_axis_name="s")

    @functools.partial(
        pl.kernel, mesh=mesh,
        out_type=jax.ShapeDtypeStruct((NQ * K_TOP, SUB), jnp.float32),
        scratch_types=[
            pltpu.VMEM((per_w,), jnp.int32),
            pltpu.VMEM((chunk, SUB), jnp.float32),
            pltpu.SemaphoreType.DMA,
        ],
    )
    def k(s3_hbm, bidx_hbm, out_hbm, idx_v, rows_v, sem):
        wid = lax.axis_index("s") * info.num_cores + lax.axis_index("c")
        base = wid * per_w
        pltpu.sync_copy(bidx_hbm.at[pl.ds(base, per_w)], idx_v)

        def to_rows(kk, _):
            off = pl.multiple_of(kk * 16, 16)
            q = wid * q_per_w + (kk >> 2)
            idx_v[pl.ds(off, 16)] = idx_v[pl.ds(off, 16)] * NQ + q
            return 0

        lax.fori_loop(0, per_w // 16, to_rows, 0)
        for c in range(nchunk):
            pltpu.async_copy(
                s3_hbm.at[idx_v.at[pl.ds(c * chunk, chunk)]], rows_v, sem
            ).wait()
            pltpu.sync_copy(rows_v, out_hbm.at[pl.ds(base + c * chunk, chunk)])

    return k(s3flat, bidx_flat)


def _p4_kernel(cand_ref, g_ref, val_ref, idx_ref, scr_ref):
    scr_ref[...] = cand_ref[...]
    g = g_ref[...]
    rows = cand_ref.shape[0]
    col64 = jax.lax.broadcasted_iota(jnp.int32, (rows, K_TOP), 1)

    def body(r, carry):
        vacc, iacc = carry
        data = scr_ref[...]
        m = jnp.max(data, axis=1, keepdims=True)
        gsel = jnp.min(jnp.where(data == m, g, BIG), axis=1, keepdims=True)
        scr_ref[...] = jnp.where(g == gsel, NEG, data)
        vacc = jnp.where(col64 == r, m, vacc)
        iacc = jnp.where(col64 == r, gsel, iacc)
        return vacc, iacc

    vacc, iacc = jax.lax.fori_loop(
        0, K_TOP, body,
        (jnp.zeros((rows, K_TOP), jnp.float32), jnp.zeros((rows, K_TOP), jnp.int32)))
    val_ref[...] = vacc
    idx_ref[...] = iacc


def kernel(z_cell, type_embeddings):
    te = jnp.pad(type_embeddings, ((0, KPAD - KEYS), (0, 0)))

    s3, bm = pl.pallas_call(
        _p1_kernel,
        grid=(NSTEP,),
        in_specs=[pl.BlockSpec((NQ, 16), lambda i: (0, 0)),
                  pl.BlockSpec((KBLK, 16), lambda i: (i, 0))],
        out_specs=[pl.BlockSpec((CPB, NQ, SUB), lambda i: (i, 0, 0)),
                   pl.BlockSpec((1, NQ, CPB), lambda i: (i, 0, 0))],
        out_shape=[jax.ShapeDtypeStruct((NB, NQ, SUB), jnp.float32),
                   jax.ShapeDtypeStruct((NSTEP, NQ, CPB), jnp.float32)],
    )(z_cell, te)
    bm = bm.transpose(1, 0, 2).reshape(NQ, NB)

    qrows = 256
    bidx, thr = pl.pallas_call(
        _p2_kernel,
        grid=(NQ // qrows,),
        in_specs=[pl.BlockSpec((qrows, NB), lambda i: (i, 0))],
        out_specs=[pl.BlockSpec((qrows, K_TOP), lambda i: (i, 0)),
                   pl.BlockSpec((qrows, 1), lambda i: (i, 0))],
        out_shape=[jax.ShapeDtypeStruct((NQ, K_TOP), jnp.int32),
                   jax.ShapeDtypeStruct((NQ, 1), jnp.float32)],
        scratch_shapes=[pltpu.VMEM((qrows, NB), jnp.float32)],
    )(bm)

    thr16 = jnp.broadcast_to(thr, (NQ, 16))
    gb16 = jnp.broadcast_to((bidx * SUB).reshape(NQ * K_TOP, 1),
                            (NQ * K_TOP, 16))
    cv, cg = _sc_gather_filter(s3.reshape(NB * NQ, SUB),
                               bidx.reshape(NQ * K_TOP), thr16, gb16)

    prows = 256
    vals, idxs = pl.pallas_call(
        _p4_kernel,
        grid=(NQ // prows,),
        in_specs=[pl.BlockSpec((prows, CAP), lambda i: (i, 0)),
                  pl.BlockSpec((prows, CAP), lambda i: (i, 0))],
        out_specs=[pl.BlockSpec((prows, K_TOP), lambda i: (i, 0)),
                   pl.BlockSpec((prows, K_TOP), lambda i: (i, 0))],
        out_shape=[jax.ShapeDtypeStruct((NQ, K_TOP), jnp.float32),
                   jax.ShapeDtypeStruct((NQ, K_TOP), jnp.int32)],
        scratch_shapes=[pltpu.VMEM((prows, CAP), jnp.float32)],
    )(cv, cg)

    return vals, idxs
